# baseline (speedup 1.0000x reference)
import functools

import jax
import jax.numpy as jnp
from jax.experimental import pallas as pl
from jax.experimental.pallas import tpu as pltpu

_LANES = 128


def _chamfer_body(xa_ref, yat_ref, loss_ref, rowacc_ref, *,
                  nj, nchunks, inv_bn, inv_bm):
    b = pl.program_id(0)
    j = pl.program_id(1)

    d2 = jax.lax.dot_general(
        xa_ref[0], yat_ref[0], (((1,), (0,)), ((), ())),
        preferred_element_type=jnp.float32)

    racc = d2[:, :_LANES]
    csum = jnp.float32(0.0)
    for c in range(nchunks):
        s = d2[:, c * _LANES:(c + 1) * _LANES]
        if c > 0:
            racc = jnp.minimum(racc, s)
        colp = jnp.min(s, axis=0, keepdims=True)
        csum = csum + jnp.sum(jnp.maximum(colp, 0.0))

    @pl.when(j == 0)
    def _init_rows():
        rowacc_ref[...] = racc

    @pl.when(j > 0)
    def _acc_rows():
        rowacc_ref[...] = jnp.minimum(rowacc_ref[...], racc)

    @pl.when((b == 0) & (j == 0))
    def _init_loss():
        loss_ref[...] = jnp.zeros_like(loss_ref)

    loss_ref[...] += csum * inv_bm

    @pl.when(j == nj - 1)
    def _flush_rows():
        rowmin = jnp.min(rowacc_ref[...], axis=1, keepdims=True)
        loss_ref[...] += (
            jnp.sum(jnp.maximum(rowmin, 0.0), keepdims=True) * inv_bn)


def kernel(pred_points, gt_points):
    x = pred_points.astype(jnp.float32)
    y = gt_points.astype(jnp.float32)
    B, N, D = x.shape
    M = y.shape[1]

    x2 = jnp.sum(x * x, axis=-1, keepdims=True)
    y2 = jnp.sum(y * y, axis=-1, keepdims=True)
    x2h = x2.astype(jnp.bfloat16).astype(jnp.float32)
    y2h = y2.astype(jnp.bfloat16).astype(jnp.float32)
    ones = jnp.ones_like(x2)
    zeros = jnp.zeros_like(x2)
    xa = jnp.concatenate(
        [-2.0 * x, x2h, x2 - x2h, ones, ones, zeros], axis=-1)
    ya = jnp.concatenate(
        [y, jnp.ones_like(y2), jnp.ones_like(y2), y2h, y2 - y2h,
         jnp.zeros_like(y2)], axis=-1)
    yat = ya.transpose(0, 2, 1)

    MT = 512 if M % 512 == 0 else M
    nj = M // MT

    out = pl.pallas_call(
        functools.partial(
            _chamfer_body, nj=nj, nchunks=MT // _LANES,
            inv_bn=1.0 / (B * N), inv_bm=1.0 / (B * M)),
        grid=(B, nj),
        in_specs=[
            pl.BlockSpec((1, N, 8), lambda b, j: (b, 0, 0)),
            pl.BlockSpec((1, 8, MT), lambda b, j: (b, 0, j)),
        ],
        out_specs=pl.BlockSpec((1, 1), lambda b, j: (0, 0)),
        out_shape=jax.ShapeDtypeStruct((1, 1), jnp.float32),
        scratch_shapes=[pltpu.VMEM((N, _LANES), jnp.float32)],
    )(xa, yat)
    return out[0, 0]


# speedup vs baseline: 1.1351x; 1.1351x over previous
import functools

import jax
import jax.numpy as jnp
from jax.experimental import pallas as pl
from jax.experimental.pallas import tpu as pltpu

_LANES = 128


def _chamfer_body(xa_ref, yat_ref, loss_ref, rowacc_ref, *,
                  nj, nchunks, inv_bn, inv_bm):
    b = pl.program_id(0)
    j = pl.program_id(1)

    d2 = jax.lax.dot_general(
        xa_ref[0], yat_ref[0], (((1,), (0,)), ((), ())),
        preferred_element_type=jnp.float32)

    racc = d2[:, :_LANES]
    csum = jnp.float32(0.0)
    for c in range(nchunks):
        s = d2[:, c * _LANES:(c + 1) * _LANES]
        if c > 0:
            racc = jnp.minimum(racc, s)
        colp = jnp.min(s, axis=0, keepdims=True)
        csum = csum + jnp.sum(jnp.maximum(colp, 0.0))

    @pl.when(j == 0)
    def _init_rows():
        rowacc_ref[...] = racc

    @pl.when(j > 0)
    def _acc_rows():
        rowacc_ref[...] = jnp.minimum(rowacc_ref[...], racc)

    @pl.when((b == 0) & (j == 0))
    def _init_loss():
        loss_ref[...] = jnp.zeros_like(loss_ref)

    loss_ref[...] += csum * inv_bm

    @pl.when(j == nj - 1)
    def _flush_rows():
        rowmin = jnp.min(rowacc_ref[...], axis=1, keepdims=True)
        loss_ref[...] += (
            jnp.sum(jnp.maximum(rowmin, 0.0), keepdims=True) * inv_bn)


def kernel(pred_points, gt_points):
    x = pred_points.astype(jnp.float32)
    y = gt_points.astype(jnp.float32)
    B, N, D = x.shape
    M = y.shape[1]

    x2 = jnp.sum(x * x, axis=-1, keepdims=True)
    y2 = jnp.sum(y * y, axis=-1, keepdims=True)
    x2h = x2.astype(jnp.bfloat16).astype(jnp.float32)
    y2h = y2.astype(jnp.bfloat16).astype(jnp.float32)
    ones = jnp.ones_like(x2)
    zeros = jnp.zeros_like(x2)
    xa = jnp.concatenate(
        [-2.0 * x, x2h, x2 - x2h, ones, ones, zeros], axis=-1)
    ya = jnp.concatenate(
        [y, jnp.ones_like(y2), jnp.ones_like(y2), y2h, y2 - y2h,
         jnp.zeros_like(y2)], axis=-1)
    yat = ya.transpose(0, 2, 1)

    xa = xa.astype(jnp.bfloat16)
    yat = yat.astype(jnp.bfloat16)

    MT = 1024 if M % 1024 == 0 else M
    nj = M // MT

    out = pl.pallas_call(
        functools.partial(
            _chamfer_body, nj=nj, nchunks=MT // _LANES,
            inv_bn=1.0 / (B * N), inv_bm=1.0 / (B * M)),
        grid=(B, nj),
        in_specs=[
            pl.BlockSpec((1, N, 8), lambda b, j: (b, 0, 0)),
            pl.BlockSpec((1, 8, MT), lambda b, j: (b, 0, j)),
        ],
        out_specs=pl.BlockSpec((1, 1), lambda b, j: (0, 0)),
        out_shape=jax.ShapeDtypeStruct((1, 1), jnp.float32),
        scratch_shapes=[pltpu.VMEM((N, _LANES), jnp.float32)],
    )(xa, yat)
    return out[0, 0]


# speedup vs baseline: 2.1017x; 1.8516x over previous
import functools

import jax
import jax.numpy as jnp
from jax.experimental import pallas as pl
from jax.experimental.pallas import tpu as pltpu

_LANES = 128


def _chamfer_body(xa_ref, ya_ref, loss_ref, rowacc_ref, *,
                  nj, nchunks, inv_bn, inv_bm):
    b = pl.program_id(0)
    j = pl.program_id(1)

    d2 = jax.lax.dot_general(
        xa_ref[0], ya_ref[0], (((0,), (0,)), ((), ())),
        preferred_element_type=jnp.float32)

    racc = d2[:, :_LANES]
    csum = jnp.float32(0.0)
    for c in range(nchunks):
        s = d2[:, c * _LANES:(c + 1) * _LANES]
        if c > 0:
            racc = jnp.minimum(racc, s)
        colp = jnp.min(s, axis=0, keepdims=True)
        csum = csum + jnp.sum(jnp.maximum(colp, 0.0))

    @pl.when(j == 0)
    def _init_rows():
        rowacc_ref[...] = racc

    @pl.when(j > 0)
    def _acc_rows():
        rowacc_ref[...] = jnp.minimum(rowacc_ref[...], racc)

    @pl.when((b == 0) & (j == 0))
    def _init_loss():
        loss_ref[...] = jnp.zeros_like(loss_ref)

    loss_ref[...] += csum * inv_bm

    @pl.when(j == nj - 1)
    def _flush_rows():
        rowmin = jnp.min(rowacc_ref[...], axis=1, keepdims=True)
        loss_ref[...] += (
            jnp.sum(jnp.maximum(rowmin, 0.0), keepdims=True) * inv_bn)


def kernel(pred_points, gt_points):
    x = pred_points.astype(jnp.float32)
    y = gt_points.astype(jnp.float32)
    B, N, D = x.shape
    M = y.shape[1]

    x0, x1, xc2 = x[:, :, 0], x[:, :, 1], x[:, :, 2]
    y0, y1, yc2 = y[:, :, 0], y[:, :, 1], y[:, :, 2]
    x2 = x0 * x0 + x1 * x1 + xc2 * xc2
    y2 = y0 * y0 + y1 * y1 + yc2 * yc2
    x2h = x2.astype(jnp.bfloat16).astype(jnp.float32)
    y2h = y2.astype(jnp.bfloat16).astype(jnp.float32)
    one_n = jnp.ones_like(x2)
    one_m = jnp.ones_like(y2)
    xa = jnp.stack(
        [-2.0 * x0, -2.0 * x1, -2.0 * xc2, x2h, x2 - x2h,
         one_n, one_n, jnp.zeros_like(x2)], axis=1)
    ya = jnp.stack(
        [y0, y1, yc2, one_m, one_m, y2h, y2 - y2h,
         jnp.zeros_like(y2)], axis=1)
    xa = xa.astype(jnp.bfloat16)
    ya = ya.astype(jnp.bfloat16)

    MT = 1024 if M % 1024 == 0 else M
    nj = M // MT

    out = pl.pallas_call(
        functools.partial(
            _chamfer_body, nj=nj, nchunks=MT // _LANES,
            inv_bn=1.0 / (B * N), inv_bm=1.0 / (B * M)),
        grid=(B, nj),
        in_specs=[
            pl.BlockSpec((1, 8, N), lambda b, j: (b, 0, 0)),
            pl.BlockSpec((1, 8, MT), lambda b, j: (b, 0, j)),
        ],
        out_specs=pl.BlockSpec((1, 1), lambda b, j: (0, 0)),
        out_shape=jax.ShapeDtypeStruct((1, 1), jnp.float32),
        scratch_shapes=[pltpu.VMEM((N, _LANES), jnp.float32)],
    )(xa, ya)
    return out[0, 0]
